# Initial kernel scaffold; baseline (speedup 1.0000x reference)
import functools

import jax
import jax.numpy as jnp
from jax import lax
from jax.experimental import pallas as pl
from jax.experimental.pallas import tpu as pltpu
from jax.experimental.pallas import tpu_sc as plsc

N = 10000
E = 320000
NT = 10016
ROWS = 2500
RPT = 79
RPT_LAST = 51
SLICE = NT // 16


def _hsd_body(x_ref, w_ref, o_ref):
    o_ref[...] = jnp.dot(x_ref[...], w_ref[...],
                         preferred_element_type=jnp.float32)


def _combine_body(part_ref, hsd_ref, b2_ref, o_ref):
    den = part_ref[0, 0:N, 0:1] + part_ref[1, 0:N, 0:1]
    num = part_ref[0, 0:N, 1:3] + part_ref[1, 0:N, 1:3]
    h = hsd_ref[:, 0:2]
    e = hsd_ref[:, 2:3] + hsd_ref[:, 3:4]
    p_self = jnp.exp(jnp.maximum(e, 0.2 * e))
    den = den + p_self + 1e-9
    num = num + p_self * h
    o = num / den + b2_ref[...]
    m = jnp.max(o, axis=1, keepdims=True)
    z = jnp.exp(o - m)
    o_ref[...] = z / jnp.sum(z, axis=1, keepdims=True)


def _sc_edges(src_hbm, dst_hbm, hsd_hbm, zero_hbm, out_hbm,
              hsd_v, src_v, dst_v, rows_v, shared_v):
    cid = lax.axis_index("c")
    sid = lax.axis_index("s")
    wid = cid * 16 + sid

    pltpu.sync_copy(zero_hbm.at[pl.ds(sid * SLICE, SLICE)],
                    shared_v.at[pl.ds(sid * SLICE, SLICE)])

    pltpu.sync_copy(hsd_hbm, hsd_v)
    row_base = wid * RPT

    @pl.when(wid != 31)
    def _():
        pltpu.sync_copy(src_hbm.at[pl.ds(row_base, RPT)], src_v)
        pltpu.sync_copy(dst_hbm.at[pl.ds(row_base, RPT)], dst_v)

    @pl.when(wid == 31)
    def _():
        pltpu.sync_copy(src_hbm.at[pl.ds(row_base, RPT_LAST)],
                        src_v.at[pl.ds(0, RPT_LAST)])
        pltpu.sync_copy(dst_hbm.at[pl.ds(row_base, RPT_LAST)],
                        dst_v.at[pl.ds(0, RPT_LAST)])

    plsc.subcore_barrier()

    iota = lax.iota(jnp.int32, 16)
    c0 = jnp.full((16,), 0, jnp.int32)
    c1 = jnp.full((16,), 1, jnp.int32)
    c2 = jnp.full((16,), 2, jnp.int32)
    c3 = jnp.full((16,), 3, jnp.int32)

    nrow = jnp.where(wid == 31, RPT_LAST, RPT)

    def block_body(j, carry):
        for l in range(8):
            src16 = src_v[j, pl.ds(l * 16, 16)]
            dst16 = dst_v[j, pl.ds(l * 16, 16)]
            sg = plsc.load_gather(hsd_v, [src16, c2])
            dg = plsc.load_gather(hsd_v, [dst16, c3])
            h0 = plsc.load_gather(hsd_v, [src16, c0])
            h1 = plsc.load_gather(hsd_v, [src16, c1])
            e = sg + dg
            p = jnp.exp(jnp.maximum(e, 0.2 * e))
            r16 = iota + l * 16
            plsc.store_scatter(rows_v, [r16, c0], p)
            plsc.store_scatter(rows_v, [r16, c1], p * h0)
            plsc.store_scatter(rows_v, [r16, c2], p * h1)
            plsc.store_scatter(rows_v, [r16, c3], p)
        pltpu.sync_copy(rows_v, shared_v.at[dst_v.at[j]], add=True)
        return carry

    lax.fori_loop(0, nrow, block_body, 0)

    plsc.subcore_barrier()
    pltpu.sync_copy(shared_v.at[pl.ds(sid * SLICE, SLICE)],
                    out_hbm.at[cid, pl.ds(sid * SLICE, SLICE)])


_sc_edge_accum = functools.partial(
    pl.kernel,
    out_type=jax.ShapeDtypeStruct((2, NT, 4), jnp.float32),
    mesh=plsc.VectorSubcoreMesh(core_axis_name="c", subcore_axis_name="s"),
    scratch_types=[
        pltpu.VMEM((N, 4), jnp.float32),
        pltpu.VMEM((RPT, 128), jnp.int32),
        pltpu.VMEM((RPT, 128), jnp.int32),
        pltpu.VMEM((128, 4), jnp.float32),
        pltpu.VMEM_SHARED((NT, 4), jnp.float32),
    ],
)(_sc_edges)


def kernel(x, edge_index, W1, a1_src, a1_dst, b1, W2, a2_src, a2_dst, b2):
    del W1, a1_src, a1_dst, b1
    wcat = jnp.concatenate(
        [W2, W2 @ a2_src[0][:, None], W2 @ a2_dst[0][:, None]], axis=1)

    hsd = pl.pallas_call(
        _hsd_body,
        grid=(10,),
        in_specs=[pl.BlockSpec((1000, 128), lambda i: (i, 0)),
                  pl.BlockSpec((128, 4), lambda i: (0, 0))],
        out_specs=pl.BlockSpec((1000, 4), lambda i: (i, 0)),
        out_shape=jax.ShapeDtypeStruct((N, 4), jnp.float32),
    )(x, wcat)

    src2d = edge_index[0].reshape(ROWS, 128)
    dst2d = edge_index[1].reshape(ROWS, 128)
    zeros = jnp.zeros((NT, 4), jnp.float32)
    part = _sc_edge_accum(src2d, dst2d, hsd, zeros)

    out = pl.pallas_call(
        _combine_body,
        out_shape=jax.ShapeDtypeStruct((N, 2), jnp.float32),
    )(part, hsd, b2.reshape(1, 2))
    return out


# speedup vs baseline: 86.3181x; 86.3181x over previous
import functools

import jax
import jax.numpy as jnp
from jax import lax
from jax.experimental import pallas as pl
from jax.experimental.pallas import tpu as pltpu
from jax.experimental.pallas import tpu_sc as plsc

N = 10000
E = 320000
NT = 10112
NT4 = NT * 4
SL4 = NT4 // 16
ECT = 10240
EPAD = 32 * ECT - E
NBLK = ECT // 128


def _hsd_body(x_ref, w_ref, o_ref):
    o_ref[...] = jnp.dot(x_ref[...], w_ref[...],
                         preferred_element_type=jnp.float32)


def _combine_body(part_ref, hsd_ref, b2_ref, o_ref):
    den = part_ref[0, 0:N, 0:1] + part_ref[1, 0:N, 0:1]
    num = part_ref[0, 0:N, 1:3] + part_ref[1, 0:N, 1:3]
    h = hsd_ref[:, 0:2]
    e = hsd_ref[:, 2:3] + hsd_ref[:, 3:4]
    p_self = jnp.exp(jnp.maximum(e, 0.2 * e))
    den = den + p_self + 1e-9
    num = num + p_self * h
    o = num / den + b2_ref[...]
    m = jnp.max(o, axis=1, keepdims=True)
    z = jnp.exp(o - m)
    o_ref[...] = z / jnp.sum(z, axis=1, keepdims=True)


def _sc_edges(src_hbm, dst_hbm, hsd_hbm, zero_hbm, out_hbm,
              hsd_v, src_v, dst_v, pv, p0v, p1v, i0, i1, i2, shared_v):
    cid = lax.axis_index("c")
    sid = lax.axis_index("s")
    wid = cid * 16 + sid

    pltpu.sync_copy(zero_hbm.at[pl.ds(sid * SL4, SL4)],
                    shared_v.at[pl.ds(sid * SL4, SL4)])

    pltpu.sync_copy(hsd_hbm, hsd_v)
    base = wid * ECT
    pltpu.sync_copy(src_hbm.at[pl.ds(base, ECT)], src_v)
    pltpu.sync_copy(dst_hbm.at[pl.ds(base, ECT)], dst_v)

    plsc.subcore_barrier()

    def block_body(j, carry):
        for l in range(8):
            off = j * 128 + l * 16
            src16 = src_v[pl.ds(off, 16)]
            dst16 = dst_v[pl.ds(off, 16)]
            s4 = src16 * 4
            sg = plsc.load_gather(hsd_v, [s4 + 2])
            dg = plsc.load_gather(hsd_v, [dst16 * 4 + 3])
            h0 = plsc.load_gather(hsd_v, [s4])
            h1 = plsc.load_gather(hsd_v, [s4 + 1])
            e = sg + dg
            p = jnp.exp(jnp.maximum(e, 0.2 * e))
            sl = pl.ds(l * 16, 16)
            pv[sl] = p
            p0v[sl] = p * h0
            p1v[sl] = p * h1
            d4 = dst16 * 4
            i0[sl] = d4
            i1[sl] = d4 + 1
            i2[sl] = d4 + 2
        pltpu.sync_copy(pv, shared_v.at[i0], add=True)
        pltpu.sync_copy(p0v, shared_v.at[i1], add=True)
        pltpu.sync_copy(p1v, shared_v.at[i2], add=True)
        return carry

    lax.fori_loop(0, NBLK, block_body, 0)

    plsc.subcore_barrier()
    pltpu.sync_copy(shared_v.at[pl.ds(sid * SL4, SL4)],
                    out_hbm.at[pl.ds(cid * NT4 + sid * SL4, SL4)])


@functools.cache
def _sc_edge_accum():
    return functools.partial(
        pl.kernel,
        out_type=jax.ShapeDtypeStruct((2 * NT4,), jnp.float32),
        mesh=plsc.VectorSubcoreMesh(core_axis_name="c", subcore_axis_name="s"),
        compiler_params=pltpu.CompilerParams(needs_layout_passes=False,
                                             use_tc_tiling_on_sc=False),
        scratch_types=[
            pltpu.VMEM((4 * N,), jnp.float32),
            pltpu.VMEM((ECT,), jnp.int32),
            pltpu.VMEM((ECT,), jnp.int32),
            pltpu.VMEM((128,), jnp.float32),
            pltpu.VMEM((128,), jnp.float32),
            pltpu.VMEM((128,), jnp.float32),
            pltpu.VMEM((128,), jnp.int32),
            pltpu.VMEM((128,), jnp.int32),
            pltpu.VMEM((128,), jnp.int32),
            pltpu.VMEM_SHARED((NT4,), jnp.float32),
        ],
    )(_sc_edges)


def kernel(x, edge_index, W1, a1_src, a1_dst, b1, W2, a2_src, a2_dst, b2):
    del W1, a1_src, a1_dst, b1
    wcat = jnp.concatenate(
        [W2, W2 @ a2_src[0][:, None], W2 @ a2_dst[0][:, None]], axis=1)

    hsd = pl.pallas_call(
        _hsd_body,
        grid=(10,),
        in_specs=[pl.BlockSpec((1000, 128), lambda i: (i, 0)),
                  pl.BlockSpec((128, 4), lambda i: (0, 0))],
        out_specs=pl.BlockSpec((1000, 4), lambda i: (i, 0)),
        out_shape=jax.ShapeDtypeStruct((N, 4), jnp.float32),
    )(x, wcat)

    src_flat = jnp.pad(edge_index[0], (0, EPAD))
    dst_flat = jnp.pad(edge_index[1], (0, EPAD), constant_values=N)
    zeros = jnp.zeros((NT4,), jnp.float32)
    part = _sc_edge_accum()(src_flat, dst_flat, hsd.reshape(4 * N), zeros)

    out = pl.pallas_call(
        _combine_body,
        out_shape=jax.ShapeDtypeStruct((N, 2), jnp.float32),
    )(part.reshape(2, NT, 4), hsd, b2.reshape(1, 2))
    return out


# speedup vs baseline: 97.1830x; 1.1259x over previous
import functools

import jax
import jax.numpy as jnp
from jax import lax
from jax.experimental import pallas as pl
from jax.experimental.pallas import tpu as pltpu
from jax.experimental.pallas import tpu_sc as plsc

N = 10000
E = 320000
ET = E + N
NT = 10240
ACC = 3 * NT
SL = ACC // 16
ECT = 10368
EPAD = 32 * ECT - ET
NBLK = ECT // 128


def _hsd_body(x_ref, w_ref, as_ref, ad_ref, o_ref):
    h = jnp.dot(x_ref[...], w_ref[...], preferred_element_type=jnp.float32)
    s = jnp.dot(h, as_ref[...], preferred_element_type=jnp.float32)
    d = jnp.dot(h, ad_ref[...], preferred_element_type=jnp.float32)
    o_ref[...] = jnp.concatenate([h, s, d], axis=1)


def _combine_body(part_ref, b2_ref, o_ref):
    den = part_ref[0, 0] + part_ref[1, 0] + 1e-9
    o0 = (part_ref[0, 1] + part_ref[1, 1]) / den + b2_ref[0, 0]
    o1 = (part_ref[0, 2] + part_ref[1, 2]) / den + b2_ref[0, 1]
    m = jnp.maximum(o0, o1)
    z0 = jnp.exp(o0 - m)
    z1 = jnp.exp(o1 - m)
    zs = z0 + z1
    o_ref[0] = z0 / zs
    o_ref[1] = z1 / zs


def _sc_edges(src_hbm, dst_hbm, hsd_hbm, zero_hbm, out_hbm,
              hsd_v, src_v, dst_v, pv, p0v, p1v, i0, i1, i2, shared_v):
    cid = lax.axis_index("c")
    sid = lax.axis_index("s")
    wid = cid * 16 + sid

    pltpu.sync_copy(zero_hbm.at[pl.ds(sid * SL, SL)],
                    shared_v.at[pl.ds(sid * SL, SL)])

    pltpu.sync_copy(hsd_hbm, hsd_v)
    base = wid * ECT
    pltpu.sync_copy(src_hbm.at[pl.ds(base, ECT)], src_v)
    pltpu.sync_copy(dst_hbm.at[pl.ds(base, ECT)], dst_v)

    plsc.subcore_barrier()

    def block_body(j, carry):
        for l in range(8):
            off = j * 128 + l * 16
            src16 = src_v[pl.ds(off, 16)]
            dst16 = dst_v[pl.ds(off, 16)]
            s4 = src16 * 4
            sg = plsc.load_gather(hsd_v, [s4 + 2])
            dg = plsc.load_gather(hsd_v, [dst16 * 4 + 3])
            h0 = plsc.load_gather(hsd_v, [s4])
            h1 = plsc.load_gather(hsd_v, [s4 + 1])
            e = sg + dg
            p = jnp.exp(jnp.maximum(e, 0.2 * e))
            sl = pl.ds(l * 16, 16)
            pv[sl] = p
            p0v[sl] = p * h0
            p1v[sl] = p * h1
            i0[sl] = dst16
            i1[sl] = dst16 + NT
            i2[sl] = dst16 + 2 * NT
        pltpu.sync_copy(pv, shared_v.at[i0], add=True)
        pltpu.sync_copy(p0v, shared_v.at[i1], add=True)
        pltpu.sync_copy(p1v, shared_v.at[i2], add=True)
        return carry

    lax.fori_loop(0, NBLK, block_body, 0)

    plsc.subcore_barrier()
    pltpu.sync_copy(shared_v.at[pl.ds(sid * SL, SL)],
                    out_hbm.at[pl.ds(cid * ACC + sid * SL, SL)])


@functools.cache
def _sc_edge_accum():
    return functools.partial(
        pl.kernel,
        out_type=jax.ShapeDtypeStruct((2 * ACC,), jnp.float32),
        mesh=plsc.VectorSubcoreMesh(core_axis_name="c", subcore_axis_name="s"),
        compiler_params=pltpu.CompilerParams(needs_layout_passes=False,
                                             use_tc_tiling_on_sc=False),
        scratch_types=[
            pltpu.VMEM((4 * N,), jnp.float32),
            pltpu.VMEM((ECT,), jnp.int32),
            pltpu.VMEM((ECT,), jnp.int32),
            pltpu.VMEM((128,), jnp.float32),
            pltpu.VMEM((128,), jnp.float32),
            pltpu.VMEM((128,), jnp.float32),
            pltpu.VMEM((128,), jnp.int32),
            pltpu.VMEM((128,), jnp.int32),
            pltpu.VMEM((128,), jnp.int32),
            pltpu.VMEM_SHARED((ACC,), jnp.float32),
        ],
    )(_sc_edges)


def kernel(x, edge_index, W1, a1_src, a1_dst, b1, W2, a2_src, a2_dst, b2):
    del W1, a1_src, a1_dst, b1
    hsd = pl.pallas_call(
        _hsd_body,
        grid=(10,),
        in_specs=[pl.BlockSpec((1000, 128), lambda i: (i, 0)),
                  pl.BlockSpec((128, 2), lambda i: (0, 0)),
                  pl.BlockSpec((2, 1), lambda i: (0, 0)),
                  pl.BlockSpec((2, 1), lambda i: (0, 0))],
        out_specs=pl.BlockSpec((1000, 4), lambda i: (i, 0)),
        out_shape=jax.ShapeDtypeStruct((N, 4), jnp.float32),
    )(x, W2, a2_src.reshape(2, 1), a2_dst.reshape(2, 1))

    loops = lax.iota(jnp.int32, N)
    src_flat = jnp.concatenate(
        [edge_index[0], loops, jnp.zeros((EPAD,), jnp.int32)])
    dst_flat = jnp.concatenate(
        [edge_index[1], loops, jnp.full((EPAD,), N, jnp.int32)])
    zeros = jnp.zeros((ACC,), jnp.float32)
    part = _sc_edge_accum()(src_flat, dst_flat, hsd.reshape(4 * N), zeros)

    out = pl.pallas_call(
        _combine_body,
        out_shape=jax.ShapeDtypeStruct((2, 80, 128), jnp.float32),
    )(part.reshape(2, 3, 80, 128), b2.reshape(1, 2))
    return out.reshape(2, NT).T[:N]


# speedup vs baseline: 98.5554x; 1.0141x over previous
import functools

import jax
import jax.numpy as jnp
from jax import lax
from jax.experimental import pallas as pl
from jax.experimental.pallas import tpu as pltpu
from jax.experimental.pallas import tpu_sc as plsc

N = 10000
E = 320000
ET = E + N
NT = 10240
ACC = 3 * NT
SL = ACC // 16
ECT = 10368
EPAD = 32 * ECT - ET
NBLK = ECT // 128


def _hsd_body(x_ref, w_ref, as_ref, ad_ref, o_ref):
    w = w_ref[...]
    w4 = jnp.concatenate(
        [w,
         jnp.dot(w, as_ref[...], preferred_element_type=jnp.float32),
         jnp.dot(w, ad_ref[...], preferred_element_type=jnp.float32)],
        axis=1)
    o_ref[...] = jnp.dot(x_ref[...], w4, preferred_element_type=jnp.float32)


def _combine_body(part_ref, b2_ref, o_ref):
    den = part_ref[0, 0] + part_ref[1, 0] + 1e-9
    o0 = (part_ref[0, 1] + part_ref[1, 1]) / den + b2_ref[0, 0]
    o1 = (part_ref[0, 2] + part_ref[1, 2]) / den + b2_ref[0, 1]
    m = jnp.maximum(o0, o1)
    z0 = jnp.exp(o0 - m)
    z1 = jnp.exp(o1 - m)
    zs = z0 + z1
    o_ref[0] = z0 / zs
    o_ref[1] = z1 / zs


def _sc_edges(src_hbm, dst_hbm, hsd_hbm, zero_hbm, out_hbm,
              hsd_v, src_v, dst_v, pv, p0v, p1v, i0, i1, i2, shared_v):
    cid = lax.axis_index("c")
    sid = lax.axis_index("s")
    wid = cid * 16 + sid

    pltpu.sync_copy(zero_hbm.at[pl.ds(sid * SL, SL)],
                    shared_v.at[pl.ds(sid * SL, SL)])

    pltpu.sync_copy(hsd_hbm, hsd_v)
    base = wid * ECT
    pltpu.sync_copy(src_hbm.at[pl.ds(base, ECT)], src_v)
    pltpu.sync_copy(dst_hbm.at[pl.ds(base, ECT)], dst_v)

    plsc.subcore_barrier()

    def block_body(j, carry):
        for l in range(8):
            off = j * 128 + l * 16
            src16 = src_v[pl.ds(off, 16)]
            dst16 = dst_v[pl.ds(off, 16)]
            s4 = src16 * 4
            sg = plsc.load_gather(hsd_v, [s4 + 2])
            dg = plsc.load_gather(hsd_v, [dst16 * 4 + 3])
            h0 = plsc.load_gather(hsd_v, [s4])
            h1 = plsc.load_gather(hsd_v, [s4 + 1])
            e = sg + dg
            p = jnp.exp(jnp.maximum(e, 0.2 * e))
            sl = pl.ds(l * 16, 16)
            pv[sl] = p
            p0v[sl] = p * h0
            p1v[sl] = p * h1
            i0[sl] = dst16
            i1[sl] = dst16 + NT
            i2[sl] = dst16 + 2 * NT
        pltpu.sync_copy(pv, shared_v.at[i0], add=True)
        pltpu.sync_copy(p0v, shared_v.at[i1], add=True)
        pltpu.sync_copy(p1v, shared_v.at[i2], add=True)
        return carry

    lax.fori_loop(0, NBLK, block_body, 0)

    plsc.subcore_barrier()
    pltpu.sync_copy(shared_v.at[pl.ds(sid * SL, SL)],
                    out_hbm.at[pl.ds(cid * ACC + sid * SL, SL)])


@functools.cache
def _sc_edge_accum():
    return functools.partial(
        pl.kernel,
        out_type=jax.ShapeDtypeStruct((2 * ACC,), jnp.float32),
        mesh=plsc.VectorSubcoreMesh(core_axis_name="c", subcore_axis_name="s"),
        compiler_params=pltpu.CompilerParams(needs_layout_passes=False,
                                             use_tc_tiling_on_sc=False),
        scratch_types=[
            pltpu.VMEM((4 * N,), jnp.float32),
            pltpu.VMEM((ECT,), jnp.int32),
            pltpu.VMEM((ECT,), jnp.int32),
            pltpu.VMEM((128,), jnp.float32),
            pltpu.VMEM((128,), jnp.float32),
            pltpu.VMEM((128,), jnp.float32),
            pltpu.VMEM((128,), jnp.int32),
            pltpu.VMEM((128,), jnp.int32),
            pltpu.VMEM((128,), jnp.int32),
            pltpu.VMEM_SHARED((ACC,), jnp.float32),
        ],
    )(_sc_edges)


def kernel(x, edge_index, W1, a1_src, a1_dst, b1, W2, a2_src, a2_dst, b2):
    del W1, a1_src, a1_dst, b1
    hsd = pl.pallas_call(
        _hsd_body,
        grid=(10,),
        in_specs=[pl.BlockSpec((1000, 128), lambda i: (i, 0)),
                  pl.BlockSpec((128, 2), lambda i: (0, 0)),
                  pl.BlockSpec((2, 1), lambda i: (0, 0)),
                  pl.BlockSpec((2, 1), lambda i: (0, 0))],
        out_specs=pl.BlockSpec((1000, 4), lambda i: (i, 0)),
        out_shape=jax.ShapeDtypeStruct((N, 4), jnp.float32),
    )(x, W2, a2_src.reshape(2, 1), a2_dst.reshape(2, 1))

    loops = lax.iota(jnp.int32, N)
    src_tail = jnp.concatenate([loops, jnp.zeros((EPAD,), jnp.int32)])
    dst_tail = jnp.concatenate([loops, jnp.full((EPAD,), N, jnp.int32)])
    src_flat = jnp.concatenate([edge_index[0], src_tail])
    dst_flat = jnp.concatenate([edge_index[1], dst_tail])
    zeros = jnp.zeros((ACC,), jnp.float32)
    part = _sc_edge_accum()(src_flat, dst_flat, hsd.reshape(4 * N), zeros)

    out = pl.pallas_call(
        _combine_body,
        out_shape=jax.ShapeDtypeStruct((2, 80, 128), jnp.float32),
    )(part.reshape(2, 3, 80, 128), b2.reshape(1, 2))
    return out.reshape(2, NT).T[:N]
